# Initial kernel scaffold; baseline (speedup 1.0000x reference)
import jax
import jax.numpy as jnp
from jax.experimental import pallas as pl


def kernel(boxes, classification, rotation, translation, hand):
    raise NotImplementedError("write your pallas kernel here")


# speedup vs baseline: 1.7828x; 1.7828x over previous
import functools

import jax
import jax.numpy as jnp
from jax import lax
from jax.experimental import pallas as pl
from jax.experimental.pallas import tpu as pltpu
from jax.experimental.pallas import tpu_sc as plsc

SCORE_THRESHOLD = 0.01
NMS_THRESHOLD = 0.5
MAX_DETECTIONS = 100
NEG = -1e30


def _nms_body(cls_ref, bx_ref, idx_out, lab_out, valid_out, score_out,
              active_ref):
    C, NPAD = cls_ref.shape
    cls = cls_ref[...]
    active_ref[...] = jnp.where(cls > SCORE_THRESHOLD, cls, NEG)

    x1 = bx_ref[0:1, :]
    y1 = bx_ref[1:2, :]
    x2 = bx_ref[2:3, :]
    y2 = bx_ref[3:4, :]
    areas = (x2 - x1) * (y2 - y1)
    lane_n = lax.broadcasted_iota(jnp.int32, (C, NPAD), 1)
    BIG = jnp.int32(2**30)

    kept_idx0 = jnp.zeros((C, 128), jnp.int32)
    kept_val0 = jnp.zeros((C, 128), jnp.int32)
    kept_sc0 = jnp.full((C, 128), NEG, jnp.float32)
    lane_k = lax.broadcasted_iota(jnp.int32, (C, 128), 1)

    def nms_iter(it, carry):
        kept_idx, kept_val, kept_sc = carry
        act = active_ref[...]
        m = jnp.max(act, axis=1, keepdims=True)
        ismax = act == m
        isel = jnp.min(jnp.where(ismax, lane_n, BIG), axis=1, keepdims=True)
        onehot = lane_n == isel
        valid = m > -1e20
        x1s = jnp.max(jnp.where(onehot, x1, NEG), axis=1, keepdims=True)
        y1s = jnp.max(jnp.where(onehot, y1, NEG), axis=1, keepdims=True)
        x2s = jnp.max(jnp.where(onehot, x2, NEG), axis=1, keepdims=True)
        y2s = jnp.max(jnp.where(onehot, y2, NEG), axis=1, keepdims=True)
        area_s = (x2s - x1s) * (y2s - y1s)
        xx1 = jnp.maximum(x1s, x1)
        yy1 = jnp.maximum(y1s, y1)
        xx2 = jnp.minimum(x2s, x2)
        yy2 = jnp.minimum(y2s, y2)
        w = jnp.maximum(0.0, xx2 - xx1)
        h = jnp.maximum(0.0, yy2 - yy1)
        inter = w * h
        iou = inter / (area_s + areas - inter + 1e-9)
        suppress = valid & ((iou > NMS_THRESHOLD) | onehot)
        active_ref[...] = jnp.where(suppress, NEG, act)
        slot = lane_k == it
        vi32 = valid.astype(jnp.int32)
        kept_idx = jnp.where(slot, jnp.where(valid, isel, 0), kept_idx)
        kept_val = jnp.where(slot, vi32, kept_val)
        kept_sc = jnp.where(slot, m, kept_sc)
        return kept_idx, kept_val, kept_sc

    kept_idx, kept_val, kept_sc = lax.fori_loop(
        0, MAX_DETECTIONS, nms_iter, (kept_idx0, kept_val0, kept_sc0))

    cls_iota = lax.broadcasted_iota(jnp.int32, (C, 128), 0)
    rank = cls_iota * 128 + lane_k
    lane_o = lax.broadcasted_iota(jnp.int32, (1, 128), 1)
    z128i = jnp.zeros((1, 128), jnp.int32)
    sel0 = (z128i, z128i, z128i, jnp.full((1, 128), NEG, jnp.float32), kept_sc)

    def pick_iter(t, carry):
        sidx, slab, sval, ssc, ks = carry
        m = jnp.max(ks)
        r0 = jnp.min(jnp.where(ks == m, rank, BIG))
        onehot = rank == r0
        iv = jnp.max(jnp.where(onehot, kept_idx, -1))
        lv = jnp.max(jnp.where(onehot, cls_iota, -1))
        vv = jnp.max(jnp.where(onehot, kept_val, -1))
        ks = jnp.where(onehot, NEG, ks)
        slot = lane_o == t
        sidx = jnp.where(slot, iv, sidx)
        slab = jnp.where(slot, lv, slab)
        sval = jnp.where(slot, vv, sval)
        ssc = jnp.where(slot, m, ssc)
        return sidx, slab, sval, ssc, ks

    sidx, slab, sval, ssc, _ = lax.fori_loop(0, MAX_DETECTIONS, pick_iter, sel0)
    idx_out[...] = sidx
    lab_out[...] = slab
    valid_out[...] = sval
    score_out[...] = ssc


def _run_nms(cls_t, bx):
    C, NPAD = cls_t.shape
    return pl.pallas_call(
        _nms_body,
        out_shape=[
            jax.ShapeDtypeStruct((1, 128), jnp.int32),
            jax.ShapeDtypeStruct((1, 128), jnp.int32),
            jax.ShapeDtypeStruct((1, 128), jnp.int32),
            jax.ShapeDtypeStruct((1, 128), jnp.float32),
        ],
        scratch_shapes=[pltpu.VMEM((C, NPAD), jnp.float32)],
    )(cls_t, bx)


def _make_sc_gather(n, d_feat):
    info = plsc.get_sparse_core_info()
    nc = info.num_cores
    mesh = plsc.VectorSubcoreMesh(core_axis_name="c", subcore_axis_name="s")

    @functools.partial(
        pl.kernel, mesh=mesh,
        compiler_params=pltpu.CompilerParams(use_tc_tiling_on_sc=False),
        out_type=jax.ShapeDtypeStruct((128, d_feat), jnp.float32),
        scratch_types=[
            pltpu.VMEM((32,), jnp.int32),
            pltpu.VMEM((32, d_feat), jnp.float32),
            pltpu.SemaphoreType.DMA,
        ],
    )
    def gather_k(idx_hbm, feat_hbm, out_feat, idx_v, feat_v, sem):
        wid = lax.axis_index("s") * nc + lax.axis_index("c")

        @pl.when(wid < 4)
        def _():
            pltpu.sync_copy(idx_hbm.at[pl.ds(wid * 32, 32)], idx_v)
            pltpu.async_copy(feat_hbm.at[idx_v], feat_v, sem).wait()
            pltpu.sync_copy(feat_v, out_feat.at[pl.ds(wid * 32, 32)])

    return gather_k


def kernel(boxes, classification, rotation, translation, hand):
    b = boxes[0]
    c = classification[0]
    r = rotation[0]
    t = translation[0]
    h = hand[0]
    n, nclass = c.shape
    npad = ((n + 127) // 128) * 128

    cls_t = jnp.pad(c.T, ((0, 0), (0, npad - n)), constant_values=-1.0)
    bx = jnp.pad(b.T, ((0, 8 - b.shape[1]), (0, npad - n)))

    sidx, slab, sval, ssc = _run_nms(cls_t, bx)
    idx128 = sidx[0]

    db, dr, dt, dh = b.shape[1], r.shape[1], t.shape[1], h.shape[1]
    d_used = db + dr + dt + dh
    d_feat = ((d_used + 15) // 16) * 16
    feat = jnp.concatenate(
        [b, r, t, h, jnp.zeros((n, d_feat - d_used), jnp.float32)], axis=1)
    g = _make_sc_gather(n, d_feat)(idx128, feat)

    m = MAX_DETECTIONS
    valid = sval[0, :m] > 0
    out_boxes = jnp.where(valid[:, None], g[:m, :db], -1.0)
    out_scores = jnp.where(valid, ssc[0, :m], -1.0)
    out_labels = jnp.where(valid, slab[0, :m], -1).astype(jnp.int32)
    out_rot = jnp.where(valid[:, None], g[:m, db:db + dr], -1.0)
    out_tr = jnp.where(valid[:, None], g[:m, db + dr:db + dr + dt], -1.0)
    out_hand = jnp.where(valid[:, None], g[:m, db + dr + dt:d_used], -1.0)
    return (out_boxes, out_scores, out_labels, out_rot, out_tr, out_hand)


# speedup vs baseline: 2.6674x; 1.4962x over previous
import functools

import jax
import jax.numpy as jnp
from jax import lax
from jax.experimental import pallas as pl
from jax.experimental.pallas import tpu as pltpu
from jax.experimental.pallas import tpu_sc as plsc

SCORE_THRESHOLD = 0.01
NMS_THRESHOLD = 0.5
MAX_DETECTIONS = 100
NEG = -1e30
CAP = 512
HOTW = 2 * CAP


def _greedy(active_ref, x1, y1, x2, y2, idx_of_lane, n_classes, width):
    C, W = n_classes, width
    areas = (x2 - x1) * (y2 - y1)
    lane = lax.broadcasted_iota(jnp.int32, (C, W), 1)
    lane_k = lax.broadcasted_iota(jnp.int32, (C, 128), 1)
    BIG = jnp.int32(2**30)
    kept0 = (jnp.zeros((C, 128), jnp.int32), jnp.zeros((C, 128), jnp.int32),
             jnp.full((C, 128), NEG, jnp.float32))

    def nms_iter(it, carry):
        kept_idx, kept_val, kept_sc = carry
        act = active_ref[...]
        m = jnp.max(act, axis=1, keepdims=True)
        ismax = act == m
        psel = jnp.min(jnp.where(ismax, lane, BIG), axis=1, keepdims=True)
        onehot = lane == psel
        valid = m > -1e20
        x1s = jnp.max(jnp.where(onehot, x1, NEG), axis=1, keepdims=True)
        y1s = jnp.max(jnp.where(onehot, y1, NEG), axis=1, keepdims=True)
        x2s = jnp.max(jnp.where(onehot, x2, NEG), axis=1, keepdims=True)
        y2s = jnp.max(jnp.where(onehot, y2, NEG), axis=1, keepdims=True)
        isel = jnp.max(jnp.where(onehot, idx_of_lane, -1), axis=1,
                       keepdims=True)
        area_s = (x2s - x1s) * (y2s - y1s)
        w = jnp.maximum(0.0, jnp.minimum(x2s, x2) - jnp.maximum(x1s, x1))
        h = jnp.maximum(0.0, jnp.minimum(y2s, y2) - jnp.maximum(y1s, y1))
        inter = w * h
        iou = inter / (area_s + areas - inter + 1e-9)
        suppress = valid & ((iou > NMS_THRESHOLD) | onehot)
        active_ref[...] = jnp.where(suppress, NEG, act)
        slot = lane_k == it
        kept_idx = jnp.where(slot, jnp.where(valid, isel, 0), kept_idx)
        kept_val = jnp.where(slot, valid.astype(jnp.int32), kept_val)
        kept_sc = jnp.where(slot, m, kept_sc)
        return kept_idx, kept_val, kept_sc

    return lax.fori_loop(0, MAX_DETECTIONS, nms_iter, kept0)


def _merge(kept_idx, kept_val, kept_sc, idx_out, lab_out, valid_out,
           score_out):
    C = kept_idx.shape[0]
    lane_k = lax.broadcasted_iota(jnp.int32, (C, 128), 1)
    cls_iota = lax.broadcasted_iota(jnp.int32, (C, 128), 0)
    rank = cls_iota * 128 + lane_k
    lane_o = lax.broadcasted_iota(jnp.int32, (1, 128), 1)
    BIG = jnp.int32(2**30)
    z = jnp.zeros((1, 128), jnp.int32)
    sel0 = (z, z, z, jnp.full((1, 128), NEG, jnp.float32), kept_sc)

    def pick_iter(t, carry):
        sidx, slab, sval, ssc, ks = carry
        m = jnp.max(ks)
        r0 = jnp.min(jnp.where(ks == m, rank, BIG))
        onehot = rank == r0
        iv = jnp.max(jnp.where(onehot, kept_idx, -1))
        lv = jnp.max(jnp.where(onehot, cls_iota, -1))
        vv = jnp.max(jnp.where(onehot, kept_val, -1))
        ks = jnp.where(onehot, NEG, ks)
        slot = lane_o == t
        return (jnp.where(slot, iv, sidx), jnp.where(slot, lv, slab),
                jnp.where(slot, vv, sval), jnp.where(slot, m, ssc), ks)

    sidx, slab, sval, ssc, _ = lax.fori_loop(0, MAX_DETECTIONS, pick_iter,
                                             sel0)
    idx_out[...] = sidx
    lab_out[...] = slab
    valid_out[...] = sval
    score_out[...] = ssc


def _tau_body(cls_ref, tau_out, cnt_out, act_ref):
    C, NPAD = cls_ref.shape
    cls = cls_ref[...]
    act = jnp.where(cls > SCORE_THRESHOLD, cls, NEG)
    act_ref[...] = act
    cntall = jnp.sum((act > -1e20).astype(jnp.int32), axis=1, keepdims=True)
    mx = jnp.max(act, axis=1, keepdims=True)
    lo0 = jnp.full((C, 1), SCORE_THRESHOLD, jnp.float32)
    hi0 = jnp.maximum(mx, lo0)

    def step(_, carry):
        lo, hi = carry
        mid = 0.5 * (lo + hi)
        a = act_ref[...]
        cnt = jnp.sum((a > mid).astype(jnp.int32), axis=1, keepdims=True)
        over = cnt > CAP
        return jnp.where(over, mid, lo), jnp.where(over, hi, mid)

    _, hi = lax.fori_loop(0, 20, step, (lo0, hi0))
    tau_out[...] = jnp.broadcast_to(hi, (C, 128))
    cnt_out[...] = jnp.broadcast_to(cntall, (C, 128))


def _run_tau(cls_t):
    C, NPAD = cls_t.shape
    return pl.pallas_call(
        _tau_body,
        out_shape=[
            jax.ShapeDtypeStruct((C, 128), jnp.float32),
            jax.ShapeDtypeStruct((C, 128), jnp.int32),
        ],
        scratch_shapes=[pltpu.VMEM((C, NPAD), jnp.float32)],
    )(cls_t)


def _make_sc_compact(n_classes, npad):
    half = npad // 2
    nv = half // 16
    info = plsc.get_sparse_core_info()
    nc = info.num_cores
    mesh = plsc.VectorSubcoreMesh(core_axis_name="c", subcore_axis_name="s")
    C = n_classes

    @functools.partial(
        pl.kernel, mesh=mesh,
        compiler_params=pltpu.CompilerParams(use_tc_tiling_on_sc=False,
                                             needs_layout_passes=False),
        out_type=[
            jax.ShapeDtypeStruct((C, HOTW), jnp.float32),
            jax.ShapeDtypeStruct((C, HOTW), jnp.int32),
            jax.ShapeDtypeStruct((C, HOTW), jnp.float32),
            jax.ShapeDtypeStruct((C, HOTW), jnp.float32),
            jax.ShapeDtypeStruct((C, HOTW), jnp.float32),
            jax.ShapeDtypeStruct((C, HOTW), jnp.float32),
        ],
        scratch_types=[
            pltpu.VMEM((half,), jnp.float32),
            pltpu.VMEM((half,), jnp.float32),
            pltpu.VMEM((half,), jnp.float32),
            pltpu.VMEM((half,), jnp.float32),
            pltpu.VMEM((half,), jnp.float32),
            pltpu.VMEM((16,), jnp.float32),
            pltpu.VMEM((CAP,), jnp.float32),
            pltpu.VMEM((CAP,), jnp.int32),
            pltpu.VMEM((CAP,), jnp.float32),
            pltpu.VMEM((CAP,), jnp.float32),
            pltpu.VMEM((CAP,), jnp.float32),
            pltpu.VMEM((CAP,), jnp.float32),
        ],
    )
    def compact_k(cls_hbm, bx_hbm, tau_hbm,
                  hsc_hbm, hidx_hbm, hx1_hbm, hy1_hbm, hx2_hbm, hy2_hbm,
                  srow, rx1, ry1, rx2, ry2, tau_v,
                  osc, oidx, ox1, oy1, ox2, oy2):
        wid = lax.axis_index("s") * nc + lax.axis_index("c")

        @pl.when(wid < 2 * C)
        def _():
            c = wid // 2
            hf = wid % 2
            base = hf * half
            pltpu.sync_copy(cls_hbm.at[c, pl.ds(base, half)], srow)
            pltpu.sync_copy(bx_hbm.at[0, pl.ds(base, half)], rx1)
            pltpu.sync_copy(bx_hbm.at[1, pl.ds(base, half)], ry1)
            pltpu.sync_copy(bx_hbm.at[2, pl.ds(base, half)], rx2)
            pltpu.sync_copy(bx_hbm.at[3, pl.ds(base, half)], ry2)
            pltpu.sync_copy(tau_hbm, tau_v)

            iota16 = lax.iota(jnp.int32, 16)
            zf = jnp.zeros((16,), jnp.float32)
            for k in range(CAP // 16):
                osc[pl.ds(k * 16, 16)] = zf + NEG
                oidx[pl.ds(k * 16, 16)] = iota16 * 0
                ox1[pl.ds(k * 16, 16)] = zf
                oy1[pl.ds(k * 16, 16)] = zf
                ox2[pl.ds(k * 16, 16)] = zf
                oy2[pl.ds(k * 16, 16)] = zf

            tau_c = plsc.load_gather(tau_v, [iota16 * 0 + c])
            base16 = iota16 + base

            def body(j, cnt):
                o = j * 16
                s = srow[pl.ds(o, 16)]
                mask = s > tau_c
                pos = cnt + plsc.cumsum(mask.astype(jnp.int32)) - 1
                wr = mask & (pos < CAP)
                plsc.store_scatter(osc, [pos], s, mask=wr)
                plsc.store_scatter(oidx, [pos], base16 + o, mask=wr)
                plsc.store_scatter(ox1, [pos], rx1[pl.ds(o, 16)], mask=wr)
                plsc.store_scatter(oy1, [pos], ry1[pl.ds(o, 16)], mask=wr)
                plsc.store_scatter(ox2, [pos], rx2[pl.ds(o, 16)], mask=wr)
                plsc.store_scatter(oy2, [pos], ry2[pl.ds(o, 16)], mask=wr)
                return cnt + plsc.all_reduce_population_count(mask)

            lax.fori_loop(0, nv, body, jnp.zeros((16,), jnp.int32))

            hout = hf * CAP
            pltpu.sync_copy(osc, hsc_hbm.at[c, pl.ds(hout, CAP)])
            pltpu.sync_copy(oidx, hidx_hbm.at[c, pl.ds(hout, CAP)])
            pltpu.sync_copy(ox1, hx1_hbm.at[c, pl.ds(hout, CAP)])
            pltpu.sync_copy(oy1, hy1_hbm.at[c, pl.ds(hout, CAP)])
            pltpu.sync_copy(ox2, hx2_hbm.at[c, pl.ds(hout, CAP)])
            pltpu.sync_copy(oy2, hy2_hbm.at[c, pl.ds(hout, CAP)])

    return compact_k


def _hot_body(hsc_ref, hidx_ref, hx1_ref, hy1_ref, hx2_ref, hy2_ref,
              cnt_ref, cls_ref, bx_ref,
              idx_out, lab_out, valid_out, score_out, hact_ref, act_ref):
    C, NPAD = cls_ref.shape
    hact_ref[...] = hsc_ref[...]
    w_hot = jnp.sum((hsc_ref[...] > -1e20).astype(jnp.int32), axis=1,
                    keepdims=True)
    kept_hot = _greedy(hact_ref, hx1_ref[...], hy1_ref[...], hx2_ref[...],
                       hy2_ref[...], hidx_ref[...], C, HOTW)
    nkept = jnp.sum(kept_hot[1], axis=1, keepdims=True)
    need_fb = (nkept < MAX_DETECTIONS) & (cnt_ref[:, 0:1] > w_hot)
    fb = jnp.max(need_fb.astype(jnp.int32)) > 0

    def full_fn(_):
        cls = cls_ref[...]
        act_ref[...] = jnp.where(cls > SCORE_THRESHOLD, cls, NEG)
        lane = lax.broadcasted_iota(jnp.int32, (C, NPAD), 1)
        return _greedy(act_ref, bx_ref[0:1, :], bx_ref[1:2, :],
                       bx_ref[2:3, :], bx_ref[3:4, :], lane, C, NPAD)

    kept_idx, kept_val, kept_sc = lax.cond(fb, full_fn, lambda _: kept_hot,
                                           None)
    _merge(kept_idx, kept_val, kept_sc, idx_out, lab_out, valid_out,
           score_out)


def _run_hot(hsc, hidx, hx1, hy1, hx2, hy2, cnt128, cls_t, bx):
    C, NPAD = cls_t.shape
    return pl.pallas_call(
        _hot_body,
        out_shape=[
            jax.ShapeDtypeStruct((1, 128), jnp.int32),
            jax.ShapeDtypeStruct((1, 128), jnp.int32),
            jax.ShapeDtypeStruct((1, 128), jnp.int32),
            jax.ShapeDtypeStruct((1, 128), jnp.float32),
        ],
        scratch_shapes=[pltpu.VMEM((C, HOTW), jnp.float32),
                        pltpu.VMEM((C, NPAD), jnp.float32)],
    )(hsc, hidx, hx1, hy1, hx2, hy2, cnt128, cls_t, bx)


def _make_sc_gather(n, d_feat):
    info = plsc.get_sparse_core_info()
    nc = info.num_cores
    mesh = plsc.VectorSubcoreMesh(core_axis_name="c", subcore_axis_name="s")

    @functools.partial(
        pl.kernel, mesh=mesh,
        compiler_params=pltpu.CompilerParams(use_tc_tiling_on_sc=False),
        out_type=jax.ShapeDtypeStruct((128, d_feat), jnp.float32),
        scratch_types=[
            pltpu.VMEM((32,), jnp.int32),
            pltpu.VMEM((32, d_feat), jnp.float32),
            pltpu.SemaphoreType.DMA,
        ],
    )
    def gather_k(idx_hbm, feat_hbm, out_feat, idx_v, feat_v, sem):
        wid = lax.axis_index("s") * nc + lax.axis_index("c")

        @pl.when(wid < 4)
        def _():
            pltpu.sync_copy(idx_hbm.at[pl.ds(wid * 32, 32)], idx_v)
            pltpu.async_copy(feat_hbm.at[idx_v], feat_v, sem).wait()
            pltpu.sync_copy(feat_v, out_feat.at[pl.ds(wid * 32, 32)])

    return gather_k


def kernel(boxes, classification, rotation, translation, hand):
    b = boxes[0]
    c = classification[0]
    r = rotation[0]
    t = translation[0]
    h = hand[0]
    n, nclass = c.shape
    npad = ((n + 127) // 128) * 128

    cls_t = jnp.pad(c.T, ((0, 0), (0, npad - n)), constant_values=-1.0)
    bx = jnp.pad(b.T, ((0, 8 - b.shape[1]), (0, npad - n)))

    tau128, cnt128 = _run_tau(cls_t)
    tau16 = jnp.pad(tau128[:, 0], (0, 16 - nclass))
    hsc, hidx, hx1, hy1, hx2, hy2 = _make_sc_compact(nclass, npad)(
        cls_t, bx, tau16)
    sidx, slab, sval, ssc = _run_hot(hsc, hidx, hx1, hy1, hx2, hy2,
                                     cnt128, cls_t, bx)
    idx128 = sidx[0]

    db, dr, dt, dh = b.shape[1], r.shape[1], t.shape[1], h.shape[1]
    d_used = db + dr + dt + dh
    d_feat = ((d_used + 15) // 16) * 16
    feat = jnp.concatenate(
        [b, r, t, h, jnp.zeros((n, d_feat - d_used), jnp.float32)], axis=1)
    g = _make_sc_gather(n, d_feat)(idx128, feat)

    m = MAX_DETECTIONS
    valid = sval[0, :m] > 0
    out_boxes = jnp.where(valid[:, None], g[:m, :db], -1.0)
    out_scores = jnp.where(valid, ssc[0, :m], -1.0)
    out_labels = jnp.where(valid, slab[0, :m], -1).astype(jnp.int32)
    out_rot = jnp.where(valid[:, None], g[:m, db:db + dr], -1.0)
    out_tr = jnp.where(valid[:, None], g[:m, db + dr:db + dr + dt], -1.0)
    out_hand = jnp.where(valid[:, None], g[:m, db + dr + dt:d_used], -1.0)
    return (out_boxes, out_scores, out_labels, out_rot, out_tr, out_hand)


# speedup vs baseline: 2.9090x; 1.0906x over previous
import functools

import jax
import jax.numpy as jnp
from jax import lax
from jax.experimental import pallas as pl
from jax.experimental.pallas import tpu as pltpu
from jax.experimental.pallas import tpu_sc as plsc

SCORE_THRESHOLD = 0.01
NMS_THRESHOLD = 0.5
MAX_DETECTIONS = 100
NEG = -1e30
CAP = 512
HOTW = 2 * CAP


def _greedy(active_ref, x1, y1, x2, y2, idx_of_lane, n_classes, width):
    C, W = n_classes, width
    areas = (x2 - x1) * (y2 - y1)
    lane = lax.broadcasted_iota(jnp.int32, (C, W), 1)
    lane_k = lax.broadcasted_iota(jnp.int32, (C, 128), 1)
    BIG = jnp.int32(2**30)
    kept0 = (jnp.zeros((C, 128), jnp.int32), jnp.zeros((C, 128), jnp.int32),
             jnp.full((C, 128), NEG, jnp.float32))

    def nms_iter(it, carry):
        kept_idx, kept_val, kept_sc = carry
        act = active_ref[...]
        m = jnp.max(act, axis=1, keepdims=True)
        ismax = act == m
        psel = jnp.min(jnp.where(ismax, lane, BIG), axis=1, keepdims=True)
        onehot = lane == psel
        valid = m > -1e20
        x1s = jnp.max(jnp.where(onehot, x1, NEG), axis=1, keepdims=True)
        y1s = jnp.max(jnp.where(onehot, y1, NEG), axis=1, keepdims=True)
        x2s = jnp.max(jnp.where(onehot, x2, NEG), axis=1, keepdims=True)
        y2s = jnp.max(jnp.where(onehot, y2, NEG), axis=1, keepdims=True)
        isel = jnp.max(jnp.where(onehot, idx_of_lane, -1), axis=1,
                       keepdims=True)
        area_s = (x2s - x1s) * (y2s - y1s)
        w = jnp.maximum(0.0, jnp.minimum(x2s, x2) - jnp.maximum(x1s, x1))
        h = jnp.maximum(0.0, jnp.minimum(y2s, y2) - jnp.maximum(y1s, y1))
        inter = w * h
        iou = inter / (area_s + areas - inter + 1e-9)
        suppress = valid & ((iou > NMS_THRESHOLD) | onehot)
        active_ref[...] = jnp.where(suppress, NEG, act)
        slot = lane_k == it
        kept_idx = jnp.where(slot, jnp.where(valid, isel, 0), kept_idx)
        kept_val = jnp.where(slot, valid.astype(jnp.int32), kept_val)
        kept_sc = jnp.where(slot, m, kept_sc)
        return kept_idx, kept_val, kept_sc

    return lax.fori_loop(0, MAX_DETECTIONS, nms_iter, kept0)


def _merge(kept_idx, kept_val, kept_sc, idx_out, lab_out, valid_out,
           score_out):
    C = kept_idx.shape[0]
    lane_k = lax.broadcasted_iota(jnp.int32, (C, 128), 1)
    cls_iota = lax.broadcasted_iota(jnp.int32, (C, 128), 0)
    rank = cls_iota * 128 + lane_k
    lane_o = lax.broadcasted_iota(jnp.int32, (1, 128), 1)
    BIG = jnp.int32(2**30)
    z = jnp.zeros((1, 128), jnp.int32)
    sel0 = (z, jnp.full((1, 128), NEG, jnp.float32), kept_sc)

    def pick_iter(t, carry):
        srank, ssc, ks = carry
        m = jnp.max(ks)
        r0 = jnp.min(jnp.where(ks == m, rank, BIG))
        ks = jnp.where(rank == r0, NEG, ks)
        slot = lane_o == t
        return (jnp.where(slot, r0, srank), jnp.where(slot, m, ssc), ks)

    srank, ssc, _ = lax.fori_loop(0, MAX_DETECTIONS, pick_iter, sel0)
    row128 = lax.broadcasted_iota(jnp.int32, (128, 128), 0)
    acc = jnp.zeros((3, 128), jnp.float32)
    idx_hi = kept_idx // 128
    idx_lo = kept_idx - idx_hi * 128
    for cc in range(C):
        onehot = (row128 + cc * 128 == srank).astype(jnp.float32)
        vals = jnp.concatenate(
            [idx_hi[cc:cc + 1, :].astype(jnp.float32),
             idx_lo[cc:cc + 1, :].astype(jnp.float32),
             kept_val[cc:cc + 1, :].astype(jnp.float32)], axis=0)
        acc = acc + jnp.dot(vals, onehot,
                            preferred_element_type=jnp.float32)
    acc = acc + 0.5
    idx_out[...] = (acc[0:1, :].astype(jnp.int32) * 128
                    + acc[1:2, :].astype(jnp.int32))
    lab_out[...] = srank // 128
    valid_out[...] = acc[2:3, :].astype(jnp.int32)
    score_out[...] = ssc


def _tau_body(cls_ref, tau_out, cnt_out, act_ref):
    C, NPAD = cls_ref.shape
    cls = cls_ref[...]
    act = jnp.where(cls > SCORE_THRESHOLD, cls, NEG)
    act_ref[...] = act
    cntall = jnp.sum((act > -1e20).astype(jnp.int32), axis=1, keepdims=True)
    mx = jnp.max(act, axis=1, keepdims=True)
    lo0 = jnp.full((C, 1), SCORE_THRESHOLD, jnp.float32)
    hi0 = jnp.maximum(mx, lo0)

    def step(_, carry):
        lo, hi = carry
        mid = 0.5 * (lo + hi)
        a = act_ref[...]
        cnt = jnp.sum((a > mid).astype(jnp.int32), axis=1, keepdims=True)
        over = cnt > CAP
        return jnp.where(over, mid, lo), jnp.where(over, hi, mid)

    _, hi = lax.fori_loop(0, 20, step, (lo0, hi0))
    tau_out[...] = jnp.broadcast_to(hi, (C, 128))
    cnt_out[...] = jnp.broadcast_to(cntall, (C, 128))


def _run_tau(cls_t):
    C, NPAD = cls_t.shape
    return pl.pallas_call(
        _tau_body,
        out_shape=[
            jax.ShapeDtypeStruct((C, 128), jnp.float32),
            jax.ShapeDtypeStruct((C, 128), jnp.int32),
        ],
        scratch_shapes=[pltpu.VMEM((C, NPAD), jnp.float32)],
    )(cls_t)


def _make_sc_compact(n_classes, npad):
    half = npad // 2
    nv = half // 16
    info = plsc.get_sparse_core_info()
    nc = info.num_cores
    mesh = plsc.VectorSubcoreMesh(core_axis_name="c", subcore_axis_name="s")
    C = n_classes

    @functools.partial(
        pl.kernel, mesh=mesh,
        compiler_params=pltpu.CompilerParams(use_tc_tiling_on_sc=False,
                                             needs_layout_passes=False),
        out_type=[
            jax.ShapeDtypeStruct((C, HOTW), jnp.float32),
            jax.ShapeDtypeStruct((C, HOTW), jnp.int32),
            jax.ShapeDtypeStruct((C, HOTW), jnp.float32),
            jax.ShapeDtypeStruct((C, HOTW), jnp.float32),
            jax.ShapeDtypeStruct((C, HOTW), jnp.float32),
            jax.ShapeDtypeStruct((C, HOTW), jnp.float32),
        ],
        scratch_types=[
            pltpu.VMEM((half,), jnp.float32),
            pltpu.VMEM((half,), jnp.float32),
            pltpu.VMEM((half,), jnp.float32),
            pltpu.VMEM((half,), jnp.float32),
            pltpu.VMEM((half,), jnp.float32),
            pltpu.VMEM((16,), jnp.float32),
            pltpu.VMEM((CAP,), jnp.float32),
            pltpu.VMEM((CAP,), jnp.int32),
            pltpu.VMEM((CAP,), jnp.float32),
            pltpu.VMEM((CAP,), jnp.float32),
            pltpu.VMEM((CAP,), jnp.float32),
            pltpu.VMEM((CAP,), jnp.float32),
        ],
    )
    def compact_k(cls_hbm, bx_hbm, tau_hbm,
                  hsc_hbm, hidx_hbm, hx1_hbm, hy1_hbm, hx2_hbm, hy2_hbm,
                  srow, rx1, ry1, rx2, ry2, tau_v,
                  osc, oidx, ox1, oy1, ox2, oy2):
        wid = lax.axis_index("s") * nc + lax.axis_index("c")

        @pl.when(wid < 2 * C)
        def _():
            c = wid // 2
            hf = wid % 2
            base = hf * half
            pltpu.sync_copy(cls_hbm.at[c, pl.ds(base, half)], srow)
            pltpu.sync_copy(bx_hbm.at[0, pl.ds(base, half)], rx1)
            pltpu.sync_copy(bx_hbm.at[1, pl.ds(base, half)], ry1)
            pltpu.sync_copy(bx_hbm.at[2, pl.ds(base, half)], rx2)
            pltpu.sync_copy(bx_hbm.at[3, pl.ds(base, half)], ry2)
            pltpu.sync_copy(tau_hbm, tau_v)

            iota16 = lax.iota(jnp.int32, 16)
            zf = jnp.zeros((16,), jnp.float32)
            for k in range(CAP // 16):
                osc[pl.ds(k * 16, 16)] = zf + NEG
                oidx[pl.ds(k * 16, 16)] = iota16 * 0
                ox1[pl.ds(k * 16, 16)] = zf
                oy1[pl.ds(k * 16, 16)] = zf
                ox2[pl.ds(k * 16, 16)] = zf
                oy2[pl.ds(k * 16, 16)] = zf

            tau_c = plsc.load_gather(tau_v, [iota16 * 0 + c])
            base16 = iota16 + base

            def body(j, cnt):
                o = j * 16
                s = srow[pl.ds(o, 16)]
                mask = s > tau_c
                pos = cnt + plsc.cumsum(mask.astype(jnp.int32)) - 1
                wr = mask & (pos < CAP)
                plsc.store_scatter(osc, [pos], s, mask=wr)
                plsc.store_scatter(oidx, [pos], base16 + o, mask=wr)
                plsc.store_scatter(ox1, [pos], rx1[pl.ds(o, 16)], mask=wr)
                plsc.store_scatter(oy1, [pos], ry1[pl.ds(o, 16)], mask=wr)
                plsc.store_scatter(ox2, [pos], rx2[pl.ds(o, 16)], mask=wr)
                plsc.store_scatter(oy2, [pos], ry2[pl.ds(o, 16)], mask=wr)
                return cnt + plsc.all_reduce_population_count(mask)

            lax.fori_loop(0, nv, body, jnp.zeros((16,), jnp.int32))

            hout = hf * CAP
            pltpu.sync_copy(osc, hsc_hbm.at[c, pl.ds(hout, CAP)])
            pltpu.sync_copy(oidx, hidx_hbm.at[c, pl.ds(hout, CAP)])
            pltpu.sync_copy(ox1, hx1_hbm.at[c, pl.ds(hout, CAP)])
            pltpu.sync_copy(oy1, hy1_hbm.at[c, pl.ds(hout, CAP)])
            pltpu.sync_copy(ox2, hx2_hbm.at[c, pl.ds(hout, CAP)])
            pltpu.sync_copy(oy2, hy2_hbm.at[c, pl.ds(hout, CAP)])

    return compact_k


def _hot_body(hsc_ref, hidx_ref, hx1_ref, hy1_ref, hx2_ref, hy2_ref,
              cnt_ref, cls_ref, bx_ref,
              idx_out, lab_out, valid_out, score_out, hact_ref, act_ref):
    C, NPAD = cls_ref.shape
    hact_ref[...] = hsc_ref[...]
    w_hot = jnp.sum((hsc_ref[...] > -1e20).astype(jnp.int32), axis=1,
                    keepdims=True)
    kept_hot = _greedy(hact_ref, hx1_ref[...], hy1_ref[...], hx2_ref[...],
                       hy2_ref[...], hidx_ref[...], C, HOTW)
    nkept = jnp.sum(kept_hot[1], axis=1, keepdims=True)
    need_fb = (nkept < MAX_DETECTIONS) & (cnt_ref[:, 0:1] > w_hot)
    fb = jnp.max(need_fb.astype(jnp.int32)) > 0

    def full_fn(_):
        cls = cls_ref[...]
        act_ref[...] = jnp.where(cls > SCORE_THRESHOLD, cls, NEG)
        lane = lax.broadcasted_iota(jnp.int32, (C, NPAD), 1)
        return _greedy(act_ref, bx_ref[0:1, :], bx_ref[1:2, :],
                       bx_ref[2:3, :], bx_ref[3:4, :], lane, C, NPAD)

    kept_idx, kept_val, kept_sc = lax.cond(fb, full_fn, lambda _: kept_hot,
                                           None)
    _merge(kept_idx, kept_val, kept_sc, idx_out, lab_out, valid_out,
           score_out)


def _run_hot(hsc, hidx, hx1, hy1, hx2, hy2, cnt128, cls_t, bx):
    C, NPAD = cls_t.shape
    return pl.pallas_call(
        _hot_body,
        out_shape=[
            jax.ShapeDtypeStruct((1, 128), jnp.int32),
            jax.ShapeDtypeStruct((1, 128), jnp.int32),
            jax.ShapeDtypeStruct((1, 128), jnp.int32),
            jax.ShapeDtypeStruct((1, 128), jnp.float32),
        ],
        scratch_shapes=[pltpu.VMEM((C, HOTW), jnp.float32),
                        pltpu.VMEM((C, NPAD), jnp.float32)],
    )(hsc, hidx, hx1, hy1, hx2, hy2, cnt128, cls_t, bx)


def _make_sc_gather(n, d_feat):
    info = plsc.get_sparse_core_info()
    nc = info.num_cores
    mesh = plsc.VectorSubcoreMesh(core_axis_name="c", subcore_axis_name="s")

    @functools.partial(
        pl.kernel, mesh=mesh,
        compiler_params=pltpu.CompilerParams(use_tc_tiling_on_sc=False),
        out_type=jax.ShapeDtypeStruct((128, d_feat), jnp.float32),
        scratch_types=[
            pltpu.VMEM((32,), jnp.int32),
            pltpu.VMEM((32, d_feat), jnp.float32),
            pltpu.SemaphoreType.DMA,
        ],
    )
    def gather_k(idx_hbm, feat_hbm, out_feat, idx_v, feat_v, sem):
        wid = lax.axis_index("s") * nc + lax.axis_index("c")

        @pl.when(wid < 4)
        def _():
            pltpu.sync_copy(idx_hbm.at[pl.ds(wid * 32, 32)], idx_v)
            pltpu.async_copy(feat_hbm.at[idx_v], feat_v, sem).wait()
            pltpu.sync_copy(feat_v, out_feat.at[pl.ds(wid * 32, 32)])

    return gather_k


def kernel(boxes, classification, rotation, translation, hand):
    b = boxes[0]
    c = classification[0]
    r = rotation[0]
    t = translation[0]
    h = hand[0]
    n, nclass = c.shape
    npad = ((n + 127) // 128) * 128

    cls_t = jnp.pad(c.T, ((0, 0), (0, npad - n)), constant_values=-1.0)
    bx = jnp.pad(b.T, ((0, 8 - b.shape[1]), (0, npad - n)))

    tau128, cnt128 = _run_tau(cls_t)
    tau16 = jnp.pad(tau128[:, 0], (0, 16 - nclass))
    hsc, hidx, hx1, hy1, hx2, hy2 = _make_sc_compact(nclass, npad)(
        cls_t, bx, tau16)
    sidx, slab, sval, ssc = _run_hot(hsc, hidx, hx1, hy1, hx2, hy2,
                                     cnt128, cls_t, bx)
    idx128 = sidx[0]

    db, dr, dt, dh = b.shape[1], r.shape[1], t.shape[1], h.shape[1]
    d_used = db + dr + dt + dh
    d_feat = ((d_used + 15) // 16) * 16
    feat = jnp.concatenate(
        [b, r, t, h, jnp.zeros((n, d_feat - d_used), jnp.float32)], axis=1)
    g = _make_sc_gather(n, d_feat)(idx128, feat)

    m = MAX_DETECTIONS
    valid = sval[0, :m] > 0
    out_boxes = jnp.where(valid[:, None], g[:m, :db], -1.0)
    out_scores = jnp.where(valid, ssc[0, :m], -1.0)
    out_labels = jnp.where(valid, slab[0, :m], -1).astype(jnp.int32)
    out_rot = jnp.where(valid[:, None], g[:m, db:db + dr], -1.0)
    out_tr = jnp.where(valid[:, None], g[:m, db + dr:db + dr + dt], -1.0)
    out_hand = jnp.where(valid[:, None], g[:m, db + dr + dt:d_used], -1.0)
    return (out_boxes, out_scores, out_labels, out_rot, out_tr, out_hand)


# speedup vs baseline: 3.0218x; 1.0388x over previous
import functools

import jax
import jax.numpy as jnp
from jax import lax
from jax.experimental import pallas as pl
from jax.experimental.pallas import tpu as pltpu
from jax.experimental.pallas import tpu_sc as plsc

SCORE_THRESHOLD = 0.01
NMS_THRESHOLD = 0.5
MAX_DETECTIONS = 100
NEG = -1e30
CAP = 192
HOTW = 2 * CAP


def _greedy(active_ref, ext_ref, x1, y1, x2, y2, idx_of_lane, n_classes,
            width):
    C, W = n_classes, width
    areas = (x2 - x1) * (y2 - y1)
    lane = lax.broadcasted_iota(jnp.int32, (C, W), 1)
    lane_k = lax.broadcasted_iota(jnp.int32, (C, 128), 1)
    BIG = jnp.int32(2**30)
    kept0 = (jnp.zeros((C, 128), jnp.int32), jnp.zeros((C, 128), jnp.int32),
             jnp.full((C, 128), NEG, jnp.float32))

    def nms_iter(it, carry):
        kept_idx, kept_val, kept_sc = carry
        act = active_ref[...]
        m = jnp.max(act, axis=1, keepdims=True)
        ismax = act == m
        psel = jnp.min(jnp.where(ismax, lane, BIG), axis=1, keepdims=True)
        onehot = lane == psel
        valid = m > -1e20
        ext_ref[0 * C:1 * C, :] = jnp.where(onehot, x1, NEG)
        ext_ref[1 * C:2 * C, :] = jnp.where(onehot, y1, NEG)
        ext_ref[2 * C:3 * C, :] = jnp.where(onehot, x2, NEG)
        ext_ref[3 * C:4 * C, :] = jnp.where(onehot, y2, NEG)
        ext_ref[4 * C:5 * C, :] = jnp.where(
            onehot, idx_of_lane.astype(jnp.float32), NEG)
        red = jnp.max(ext_ref[...], axis=1, keepdims=True)
        x1s = red[0 * C:1 * C]
        y1s = red[1 * C:2 * C]
        x2s = red[2 * C:3 * C]
        y2s = red[3 * C:4 * C]
        isel = jnp.where(valid, red[4 * C:5 * C], 0.0).astype(jnp.int32)
        area_s = (x2s - x1s) * (y2s - y1s)
        w = jnp.maximum(0.0, jnp.minimum(x2s, x2) - jnp.maximum(x1s, x1))
        h = jnp.maximum(0.0, jnp.minimum(y2s, y2) - jnp.maximum(y1s, y1))
        inter = w * h
        iou = inter / (area_s + areas - inter + 1e-9)
        suppress = valid & ((iou > NMS_THRESHOLD) | onehot)
        active_ref[...] = jnp.where(suppress, NEG, act)
        slot = lane_k == it
        kept_idx = jnp.where(slot, jnp.where(valid, isel, 0), kept_idx)
        kept_val = jnp.where(slot, valid.astype(jnp.int32), kept_val)
        kept_sc = jnp.where(slot, m, kept_sc)
        return kept_idx, kept_val, kept_sc

    return lax.fori_loop(0, MAX_DETECTIONS, nms_iter, kept0)


def _merge(kept_idx, kept_val, kept_sc, idx_out, lab_out, valid_out,
           score_out):
    C = kept_idx.shape[0]
    lane_k = lax.broadcasted_iota(jnp.int32, (C, 128), 1)
    cls_iota = lax.broadcasted_iota(jnp.int32, (C, 128), 0)
    rank = cls_iota * 128 + lane_k
    lane_o = lax.broadcasted_iota(jnp.int32, (1, 128), 1)
    BIG = jnp.int32(2**30)
    z = jnp.zeros((1, 128), jnp.int32)
    sel0 = (z, jnp.full((1, 128), NEG, jnp.float32), kept_sc)

    def pick_iter(t, carry):
        srank, ssc, ks = carry
        m = jnp.max(ks)
        r0 = jnp.min(jnp.where(ks == m, rank, BIG))
        ks = jnp.where(rank == r0, NEG, ks)
        slot = lane_o == t
        return (jnp.where(slot, r0, srank), jnp.where(slot, m, ssc), ks)

    srank, ssc, _ = lax.fori_loop(0, MAX_DETECTIONS, pick_iter, sel0)
    row128 = lax.broadcasted_iota(jnp.int32, (128, 128), 0)
    acc = jnp.zeros((3, 128), jnp.float32)
    idx_hi = kept_idx // 128
    idx_lo = kept_idx - idx_hi * 128
    for cc in range(C):
        onehot = (row128 + cc * 128 == srank).astype(jnp.float32)
        vals = jnp.concatenate(
            [idx_hi[cc:cc + 1, :].astype(jnp.float32),
             idx_lo[cc:cc + 1, :].astype(jnp.float32),
             kept_val[cc:cc + 1, :].astype(jnp.float32)], axis=0)
        acc = acc + jnp.dot(vals, onehot,
                            preferred_element_type=jnp.float32)
    acc = acc + 0.5
    idx_out[...] = (acc[0:1, :].astype(jnp.int32) * 128
                    + acc[1:2, :].astype(jnp.int32))
    lab_out[...] = srank // 128
    valid_out[...] = acc[2:3, :].astype(jnp.int32)
    score_out[...] = ssc


def _tau_body(cls_ref, tau_out, cnt_out, act_ref):
    C, NPAD = cls_ref.shape
    cls = cls_ref[...]
    act = jnp.where(cls > SCORE_THRESHOLD, cls, NEG)
    act_ref[...] = act
    cntall = jnp.sum((act > -1e20).astype(jnp.int32), axis=1, keepdims=True)
    mx = jnp.max(act, axis=1, keepdims=True)
    lo0 = jnp.full((C, 1), SCORE_THRESHOLD, jnp.float32)
    hi0 = jnp.maximum(mx, lo0)

    def step(_, carry):
        lo, hi = carry
        mid = 0.5 * (lo + hi)
        a = act_ref[...]
        cnt = jnp.sum((a > mid).astype(jnp.int32), axis=1, keepdims=True)
        over = cnt > CAP
        return jnp.where(over, mid, lo), jnp.where(over, hi, mid)

    _, hi = lax.fori_loop(0, 20, step, (lo0, hi0))
    tau_out[...] = jnp.broadcast_to(hi, (C, 128))
    cnt_out[...] = jnp.broadcast_to(cntall, (C, 128))


def _run_tau(cls_t):
    C, NPAD = cls_t.shape
    return pl.pallas_call(
        _tau_body,
        out_shape=[
            jax.ShapeDtypeStruct((C, 128), jnp.float32),
            jax.ShapeDtypeStruct((C, 128), jnp.int32),
        ],
        scratch_shapes=[pltpu.VMEM((C, NPAD), jnp.float32)],
    )(cls_t)


def _make_sc_compact(n_classes, npad):
    half = npad // 2
    nv = half // 16
    info = plsc.get_sparse_core_info()
    nc = info.num_cores
    mesh = plsc.VectorSubcoreMesh(core_axis_name="c", subcore_axis_name="s")
    C = n_classes

    @functools.partial(
        pl.kernel, mesh=mesh,
        compiler_params=pltpu.CompilerParams(use_tc_tiling_on_sc=False,
                                             needs_layout_passes=False),
        out_type=[
            jax.ShapeDtypeStruct((C, HOTW), jnp.float32),
            jax.ShapeDtypeStruct((C, HOTW), jnp.int32),
            jax.ShapeDtypeStruct((C, HOTW), jnp.float32),
            jax.ShapeDtypeStruct((C, HOTW), jnp.float32),
            jax.ShapeDtypeStruct((C, HOTW), jnp.float32),
            jax.ShapeDtypeStruct((C, HOTW), jnp.float32),
        ],
        scratch_types=[
            pltpu.VMEM((half,), jnp.float32),
            pltpu.VMEM((half,), jnp.float32),
            pltpu.VMEM((half,), jnp.float32),
            pltpu.VMEM((half,), jnp.float32),
            pltpu.VMEM((half,), jnp.float32),
            pltpu.VMEM((16,), jnp.float32),
            pltpu.VMEM((CAP,), jnp.float32),
            pltpu.VMEM((CAP,), jnp.int32),
            pltpu.VMEM((CAP,), jnp.float32),
            pltpu.VMEM((CAP,), jnp.float32),
            pltpu.VMEM((CAP,), jnp.float32),
            pltpu.VMEM((CAP,), jnp.float32),
        ],
    )
    def compact_k(cls_hbm, bx_hbm, tau_hbm,
                  hsc_hbm, hidx_hbm, hx1_hbm, hy1_hbm, hx2_hbm, hy2_hbm,
                  srow, rx1, ry1, rx2, ry2, tau_v,
                  osc, oidx, ox1, oy1, ox2, oy2):
        wid = lax.axis_index("s") * nc + lax.axis_index("c")

        @pl.when(wid < 2 * C)
        def _():
            c = wid // 2
            hf = wid % 2
            base = hf * half
            pltpu.sync_copy(cls_hbm.at[c, pl.ds(base, half)], srow)
            pltpu.sync_copy(bx_hbm.at[0, pl.ds(base, half)], rx1)
            pltpu.sync_copy(bx_hbm.at[1, pl.ds(base, half)], ry1)
            pltpu.sync_copy(bx_hbm.at[2, pl.ds(base, half)], rx2)
            pltpu.sync_copy(bx_hbm.at[3, pl.ds(base, half)], ry2)
            pltpu.sync_copy(tau_hbm, tau_v)

            iota16 = lax.iota(jnp.int32, 16)
            zf = jnp.zeros((16,), jnp.float32)
            for k in range(CAP // 16):
                osc[pl.ds(k * 16, 16)] = zf + NEG
                oidx[pl.ds(k * 16, 16)] = iota16 * 0
                ox1[pl.ds(k * 16, 16)] = zf
                oy1[pl.ds(k * 16, 16)] = zf
                ox2[pl.ds(k * 16, 16)] = zf
                oy2[pl.ds(k * 16, 16)] = zf

            tau_c = plsc.load_gather(tau_v, [iota16 * 0 + c])
            base16 = iota16 + base

            def body(j, cnt):
                o = j * 16
                s = srow[pl.ds(o, 16)]
                mask = s > tau_c
                pos = cnt + plsc.cumsum(mask.astype(jnp.int32)) - 1
                wr = mask & (pos < CAP)
                plsc.store_scatter(osc, [pos], s, mask=wr)
                plsc.store_scatter(oidx, [pos], base16 + o, mask=wr)
                plsc.store_scatter(ox1, [pos], rx1[pl.ds(o, 16)], mask=wr)
                plsc.store_scatter(oy1, [pos], ry1[pl.ds(o, 16)], mask=wr)
                plsc.store_scatter(ox2, [pos], rx2[pl.ds(o, 16)], mask=wr)
                plsc.store_scatter(oy2, [pos], ry2[pl.ds(o, 16)], mask=wr)
                return cnt + plsc.all_reduce_population_count(mask)

            lax.fori_loop(0, nv, body, jnp.zeros((16,), jnp.int32))

            hout = hf * CAP
            pltpu.sync_copy(osc, hsc_hbm.at[c, pl.ds(hout, CAP)])
            pltpu.sync_copy(oidx, hidx_hbm.at[c, pl.ds(hout, CAP)])
            pltpu.sync_copy(ox1, hx1_hbm.at[c, pl.ds(hout, CAP)])
            pltpu.sync_copy(oy1, hy1_hbm.at[c, pl.ds(hout, CAP)])
            pltpu.sync_copy(ox2, hx2_hbm.at[c, pl.ds(hout, CAP)])
            pltpu.sync_copy(oy2, hy2_hbm.at[c, pl.ds(hout, CAP)])

    return compact_k


def _hot_body(hsc_ref, hidx_ref, hx1_ref, hy1_ref, hx2_ref, hy2_ref,
              cnt_ref, cls_ref, bx_ref,
              idx_out, lab_out, valid_out, score_out, hact_ref, act_ref,
              exth_ref, extf_ref):
    C, NPAD = cls_ref.shape
    hact_ref[...] = hsc_ref[...]
    w_hot = jnp.sum((hsc_ref[...] > -1e20).astype(jnp.int32), axis=1,
                    keepdims=True)
    kept_hot = _greedy(hact_ref, exth_ref, hx1_ref[...], hy1_ref[...],
                       hx2_ref[...], hy2_ref[...], hidx_ref[...], C, HOTW)
    nkept = jnp.sum(kept_hot[1], axis=1, keepdims=True)
    need_fb = (nkept < MAX_DETECTIONS) & (cnt_ref[:, 0:1] > w_hot)
    fb = jnp.max(need_fb.astype(jnp.int32)) > 0

    def full_fn(_):
        cls = cls_ref[...]
        act_ref[...] = jnp.where(cls > SCORE_THRESHOLD, cls, NEG)
        lane = lax.broadcasted_iota(jnp.int32, (C, NPAD), 1)
        return _greedy(act_ref, extf_ref, bx_ref[0:1, :], bx_ref[1:2, :],
                       bx_ref[2:3, :], bx_ref[3:4, :], lane, C, NPAD)

    kept_idx, kept_val, kept_sc = lax.cond(fb, full_fn, lambda _: kept_hot,
                                           None)
    _merge(kept_idx, kept_val, kept_sc, idx_out, lab_out, valid_out,
           score_out)


def _run_hot(hsc, hidx, hx1, hy1, hx2, hy2, cnt128, cls_t, bx):
    C, NPAD = cls_t.shape
    return pl.pallas_call(
        _hot_body,
        out_shape=[
            jax.ShapeDtypeStruct((1, 128), jnp.int32),
            jax.ShapeDtypeStruct((1, 128), jnp.int32),
            jax.ShapeDtypeStruct((1, 128), jnp.int32),
            jax.ShapeDtypeStruct((1, 128), jnp.float32),
        ],
        scratch_shapes=[pltpu.VMEM((C, HOTW), jnp.float32),
                        pltpu.VMEM((C, NPAD), jnp.float32),
                        pltpu.VMEM((5 * C, HOTW), jnp.float32),
                        pltpu.VMEM((5 * C, NPAD), jnp.float32)],
    )(hsc, hidx, hx1, hy1, hx2, hy2, cnt128, cls_t, bx)


def _make_sc_gather(n, d_feat):
    info = plsc.get_sparse_core_info()
    nc = info.num_cores
    mesh = plsc.VectorSubcoreMesh(core_axis_name="c", subcore_axis_name="s")

    @functools.partial(
        pl.kernel, mesh=mesh,
        compiler_params=pltpu.CompilerParams(use_tc_tiling_on_sc=False),
        out_type=jax.ShapeDtypeStruct((128, d_feat), jnp.float32),
        scratch_types=[
            pltpu.VMEM((32,), jnp.int32),
            pltpu.VMEM((32, d_feat), jnp.float32),
            pltpu.SemaphoreType.DMA,
        ],
    )
    def gather_k(idx_hbm, feat_hbm, out_feat, idx_v, feat_v, sem):
        wid = lax.axis_index("s") * nc + lax.axis_index("c")

        @pl.when(wid < 4)
        def _():
            pltpu.sync_copy(idx_hbm.at[pl.ds(wid * 32, 32)], idx_v)
            pltpu.async_copy(feat_hbm.at[idx_v], feat_v, sem).wait()
            pltpu.sync_copy(feat_v, out_feat.at[pl.ds(wid * 32, 32)])

    return gather_k


def kernel(boxes, classification, rotation, translation, hand):
    b = boxes[0]
    c = classification[0]
    r = rotation[0]
    t = translation[0]
    h = hand[0]
    n, nclass = c.shape
    npad = ((n + 127) // 128) * 128

    cls_t = jnp.pad(c.T, ((0, 0), (0, npad - n)), constant_values=-1.0)
    bx = jnp.pad(b.T, ((0, 8 - b.shape[1]), (0, npad - n)))

    tau128, cnt128 = _run_tau(cls_t)
    tau16 = jnp.pad(tau128[:, 0], (0, 16 - nclass))
    hsc, hidx, hx1, hy1, hx2, hy2 = _make_sc_compact(nclass, npad)(
        cls_t, bx, tau16)
    sidx, slab, sval, ssc = _run_hot(hsc, hidx, hx1, hy1, hx2, hy2,
                                     cnt128, cls_t, bx)
    idx128 = sidx[0]

    db, dr, dt, dh = b.shape[1], r.shape[1], t.shape[1], h.shape[1]
    d_used = db + dr + dt + dh
    d_feat = ((d_used + 15) // 16) * 16
    feat = jnp.concatenate(
        [b, r, t, h, jnp.zeros((n, d_feat - d_used), jnp.float32)], axis=1)
    g = _make_sc_gather(n, d_feat)(idx128, feat)

    m = MAX_DETECTIONS
    valid = sval[0, :m] > 0
    out_boxes = jnp.where(valid[:, None], g[:m, :db], -1.0)
    out_scores = jnp.where(valid, ssc[0, :m], -1.0)
    out_labels = jnp.where(valid, slab[0, :m], -1).astype(jnp.int32)
    out_rot = jnp.where(valid[:, None], g[:m, db:db + dr], -1.0)
    out_tr = jnp.where(valid[:, None], g[:m, db + dr:db + dr + dt], -1.0)
    out_hand = jnp.where(valid[:, None], g[:m, db + dr + dt:d_used], -1.0)
    return (out_boxes, out_scores, out_labels, out_rot, out_tr, out_hand)


# speedup vs baseline: 3.0222x; 1.0001x over previous
import functools

import jax
import jax.numpy as jnp
from jax import lax
from jax.experimental import pallas as pl
from jax.experimental.pallas import tpu as pltpu
from jax.experimental.pallas import tpu_sc as plsc

SCORE_THRESHOLD = 0.01
NMS_THRESHOLD = 0.5
MAX_DETECTIONS = 100
NEG = -1e30
CAP = 192
HOTW = 2 * CAP


def _greedy(active_ref, ext_ref, x1, y1, x2, y2, idx_of_lane, n_classes,
            width):
    C, W = n_classes, width
    areas = (x2 - x1) * (y2 - y1)
    lane = lax.broadcasted_iota(jnp.int32, (C, W), 1)
    lane_k = lax.broadcasted_iota(jnp.int32, (C, 128), 1)
    BIG = jnp.int32(2**30)
    kept0 = (jnp.zeros((C, 128), jnp.int32), jnp.zeros((C, 128), jnp.int32),
             jnp.full((C, 128), NEG, jnp.float32))

    def nms_iter(it, carry):
        kept_idx, kept_val, kept_sc = carry
        act = active_ref[...]
        m = jnp.max(act, axis=1, keepdims=True)
        ismax = act == m
        psel = jnp.min(jnp.where(ismax, lane, BIG), axis=1, keepdims=True)
        onehot = lane == psel
        valid = m > -1e20
        ext_ref[0 * C:1 * C, :] = jnp.where(onehot, x1, NEG)
        ext_ref[1 * C:2 * C, :] = jnp.where(onehot, y1, NEG)
        ext_ref[2 * C:3 * C, :] = jnp.where(onehot, x2, NEG)
        ext_ref[3 * C:4 * C, :] = jnp.where(onehot, y2, NEG)
        ext_ref[4 * C:5 * C, :] = jnp.where(
            onehot, idx_of_lane.astype(jnp.float32), NEG)
        red = jnp.max(ext_ref[...], axis=1, keepdims=True)
        x1s = red[0 * C:1 * C]
        y1s = red[1 * C:2 * C]
        x2s = red[2 * C:3 * C]
        y2s = red[3 * C:4 * C]
        isel = jnp.where(valid, red[4 * C:5 * C], 0.0).astype(jnp.int32)
        area_s = (x2s - x1s) * (y2s - y1s)
        w = jnp.maximum(0.0, jnp.minimum(x2s, x2) - jnp.maximum(x1s, x1))
        h = jnp.maximum(0.0, jnp.minimum(y2s, y2) - jnp.maximum(y1s, y1))
        inter = w * h
        iou = inter / (area_s + areas - inter + 1e-9)
        suppress = valid & ((iou > NMS_THRESHOLD) | onehot)
        active_ref[...] = jnp.where(suppress, NEG, act)
        slot = lane_k == it
        kept_idx = jnp.where(slot, jnp.where(valid, isel, 0), kept_idx)
        kept_val = jnp.where(slot, valid.astype(jnp.int32), kept_val)
        kept_sc = jnp.where(slot, m, kept_sc)
        return kept_idx, kept_val, kept_sc

    return lax.fori_loop(0, MAX_DETECTIONS, nms_iter, kept0)


def _merge(kept_idx, kept_val, kept_sc, idx_out, lab_out, valid_out,
           score_out):
    C = kept_idx.shape[0]
    lane_k = lax.broadcasted_iota(jnp.int32, (C, 128), 1)
    cls_iota = lax.broadcasted_iota(jnp.int32, (C, 128), 0)
    rank = cls_iota * 128 + lane_k
    lane_o = lax.broadcasted_iota(jnp.int32, (1, 128), 1)
    BIG = jnp.int32(2**30)
    z = jnp.zeros((1, 128), jnp.int32)
    sel0 = (z, jnp.full((1, 128), NEG, jnp.float32), kept_sc)

    def pick_iter(t, carry):
        srank, ssc, ks = carry
        m = jnp.max(ks)
        r0 = jnp.min(jnp.where(ks == m, rank, BIG))
        ks = jnp.where(rank == r0, NEG, ks)
        slot = lane_o == t
        return (jnp.where(slot, r0, srank), jnp.where(slot, m, ssc), ks)

    srank, ssc, _ = lax.fori_loop(0, MAX_DETECTIONS, pick_iter, sel0)
    row128 = lax.broadcasted_iota(jnp.int32, (128, 128), 0)
    acc = jnp.zeros((3, 128), jnp.float32)
    idx_hi = kept_idx // 128
    idx_lo = kept_idx - idx_hi * 128
    for cc in range(C):
        onehot = (row128 + cc * 128 == srank).astype(jnp.float32)
        vals = jnp.concatenate(
            [idx_hi[cc:cc + 1, :].astype(jnp.float32),
             idx_lo[cc:cc + 1, :].astype(jnp.float32),
             kept_val[cc:cc + 1, :].astype(jnp.float32)], axis=0)
        acc = acc + jnp.dot(vals, onehot,
                            preferred_element_type=jnp.float32)
    acc = acc + 0.5
    idx_out[...] = (acc[0:1, :].astype(jnp.int32) * 128
                    + acc[1:2, :].astype(jnp.int32))
    lab_out[...] = srank // 128
    valid_out[...] = acc[2:3, :].astype(jnp.int32)
    score_out[...] = ssc


def _tau_body(cls_ref, tau_out, cnt_out, act_ref):
    C, NPAD = cls_ref.shape
    cls = cls_ref[...]
    act = jnp.where(cls > SCORE_THRESHOLD, cls, NEG)
    act_ref[...] = act
    cntall = jnp.sum((act > -1e20).astype(jnp.int32), axis=1, keepdims=True)
    mx = jnp.max(act, axis=1, keepdims=True)
    lo0 = jnp.full((C, 1), SCORE_THRESHOLD, jnp.float32)
    hi0 = jnp.maximum(mx, lo0)

    def step(_, carry):
        lo, hi = carry
        mid = 0.5 * (lo + hi)
        a = act_ref[...]
        cnt = jnp.sum((a > mid).astype(jnp.int32), axis=1, keepdims=True)
        over = cnt > CAP
        return jnp.where(over, mid, lo), jnp.where(over, hi, mid)

    _, hi = lax.fori_loop(0, 20, step, (lo0, hi0))
    tau_out[...] = jnp.broadcast_to(hi, (C, 128))
    cnt_out[...] = jnp.broadcast_to(cntall, (C, 128))


def _run_tau(cls_t):
    C, NPAD = cls_t.shape
    return pl.pallas_call(
        _tau_body,
        out_shape=[
            jax.ShapeDtypeStruct((C, 128), jnp.float32),
            jax.ShapeDtypeStruct((C, 128), jnp.int32),
        ],
        scratch_shapes=[pltpu.VMEM((C, NPAD), jnp.float32)],
    )(cls_t)


def _make_sc_compact(n_classes, npad):
    half = npad // 2
    nv = half // 16
    info = plsc.get_sparse_core_info()
    nc = info.num_cores
    mesh = plsc.VectorSubcoreMesh(core_axis_name="c", subcore_axis_name="s")
    C = n_classes

    @functools.partial(
        pl.kernel, mesh=mesh,
        compiler_params=pltpu.CompilerParams(use_tc_tiling_on_sc=False,
                                             needs_layout_passes=False),
        out_type=[
            jax.ShapeDtypeStruct((C, HOTW), jnp.float32),
            jax.ShapeDtypeStruct((C, HOTW), jnp.int32),
            jax.ShapeDtypeStruct((C, HOTW), jnp.float32),
            jax.ShapeDtypeStruct((C, HOTW), jnp.float32),
            jax.ShapeDtypeStruct((C, HOTW), jnp.float32),
            jax.ShapeDtypeStruct((C, HOTW), jnp.float32),
        ],
        scratch_types=[
            pltpu.VMEM((half,), jnp.float32),
            pltpu.VMEM((half,), jnp.float32),
            pltpu.VMEM((half,), jnp.float32),
            pltpu.VMEM((half,), jnp.float32),
            pltpu.VMEM((half,), jnp.float32),
            pltpu.VMEM((16,), jnp.float32),
            pltpu.VMEM((CAP,), jnp.float32),
            pltpu.VMEM((CAP,), jnp.int32),
            pltpu.VMEM((CAP,), jnp.float32),
            pltpu.VMEM((CAP,), jnp.float32),
            pltpu.VMEM((CAP,), jnp.float32),
            pltpu.VMEM((CAP,), jnp.float32),
        ],
    )
    def compact_k(cls_hbm, bx_hbm, tau_hbm,
                  hsc_hbm, hidx_hbm, hx1_hbm, hy1_hbm, hx2_hbm, hy2_hbm,
                  srow, rx1, ry1, rx2, ry2, tau_v,
                  osc, oidx, ox1, oy1, ox2, oy2):
        wid = lax.axis_index("s") * nc + lax.axis_index("c")

        @pl.when(wid < 2 * C)
        def _():
            c = wid // 2
            hf = wid % 2
            base = hf * half
            pltpu.sync_copy(cls_hbm.at[c, pl.ds(base, half)], srow)
            pltpu.sync_copy(bx_hbm.at[0, pl.ds(base, half)], rx1)
            pltpu.sync_copy(bx_hbm.at[1, pl.ds(base, half)], ry1)
            pltpu.sync_copy(bx_hbm.at[2, pl.ds(base, half)], rx2)
            pltpu.sync_copy(bx_hbm.at[3, pl.ds(base, half)], ry2)
            pltpu.sync_copy(tau_hbm, tau_v)

            iota16 = lax.iota(jnp.int32, 16)
            zf = jnp.zeros((16,), jnp.float32)
            for k in range(CAP // 16):
                osc[pl.ds(k * 16, 16)] = zf + NEG
                oidx[pl.ds(k * 16, 16)] = iota16 * 0
                ox1[pl.ds(k * 16, 16)] = zf
                oy1[pl.ds(k * 16, 16)] = zf
                ox2[pl.ds(k * 16, 16)] = zf
                oy2[pl.ds(k * 16, 16)] = zf

            tau_c = plsc.load_gather(tau_v, [iota16 * 0 + c])
            base16 = iota16 + base

            def body(j, cnt):
                for u in range(4):
                    o = j * 64 + u * 16
                    s = srow[pl.ds(o, 16)]
                    mask = s > tau_c
                    pos = cnt + plsc.cumsum(mask.astype(jnp.int32)) - 1
                    wr = mask & (pos < CAP)
                    plsc.store_scatter(osc, [pos], s, mask=wr)
                    plsc.store_scatter(oidx, [pos], base16 + o, mask=wr)
                    plsc.store_scatter(ox1, [pos], rx1[pl.ds(o, 16)],
                                       mask=wr)
                    plsc.store_scatter(oy1, [pos], ry1[pl.ds(o, 16)],
                                       mask=wr)
                    plsc.store_scatter(ox2, [pos], rx2[pl.ds(o, 16)],
                                       mask=wr)
                    plsc.store_scatter(oy2, [pos], ry2[pl.ds(o, 16)],
                                       mask=wr)
                    cnt = cnt + plsc.all_reduce_population_count(mask)
                return cnt

            cnt = lax.fori_loop(0, nv // 4, body,
                                jnp.zeros((16,), jnp.int32))
            for jt in range(4 * (nv // 4), nv):
                o = jt * 16
                s = srow[pl.ds(o, 16)]
                mask = s > tau_c
                pos = cnt + plsc.cumsum(mask.astype(jnp.int32)) - 1
                wr = mask & (pos < CAP)
                plsc.store_scatter(osc, [pos], s, mask=wr)
                plsc.store_scatter(oidx, [pos], base16 + o, mask=wr)
                plsc.store_scatter(ox1, [pos], rx1[pl.ds(o, 16)], mask=wr)
                plsc.store_scatter(oy1, [pos], ry1[pl.ds(o, 16)], mask=wr)
                plsc.store_scatter(ox2, [pos], rx2[pl.ds(o, 16)], mask=wr)
                plsc.store_scatter(oy2, [pos], ry2[pl.ds(o, 16)], mask=wr)
                cnt = cnt + plsc.all_reduce_population_count(mask)

            hout = hf * CAP
            pltpu.sync_copy(osc, hsc_hbm.at[c, pl.ds(hout, CAP)])
            pltpu.sync_copy(oidx, hidx_hbm.at[c, pl.ds(hout, CAP)])
            pltpu.sync_copy(ox1, hx1_hbm.at[c, pl.ds(hout, CAP)])
            pltpu.sync_copy(oy1, hy1_hbm.at[c, pl.ds(hout, CAP)])
            pltpu.sync_copy(ox2, hx2_hbm.at[c, pl.ds(hout, CAP)])
            pltpu.sync_copy(oy2, hy2_hbm.at[c, pl.ds(hout, CAP)])

    return compact_k


def _hot_body(hsc_ref, hidx_ref, hx1_ref, hy1_ref, hx2_ref, hy2_ref,
              cnt_ref, cls_ref, bx_ref,
              idx_out, lab_out, valid_out, score_out, hact_ref, act_ref,
              exth_ref, extf_ref):
    C, NPAD = cls_ref.shape
    hact_ref[...] = hsc_ref[...]
    w_hot = jnp.sum((hsc_ref[...] > -1e20).astype(jnp.int32), axis=1,
                    keepdims=True)
    kept_hot = _greedy(hact_ref, exth_ref, hx1_ref[...], hy1_ref[...],
                       hx2_ref[...], hy2_ref[...], hidx_ref[...], C, HOTW)
    nkept = jnp.sum(kept_hot[1], axis=1, keepdims=True)
    need_fb = (nkept < MAX_DETECTIONS) & (cnt_ref[:, 0:1] > w_hot)
    fb = jnp.max(need_fb.astype(jnp.int32)) > 0

    def full_fn(_):
        cls = cls_ref[...]
        act_ref[...] = jnp.where(cls > SCORE_THRESHOLD, cls, NEG)
        lane = lax.broadcasted_iota(jnp.int32, (C, NPAD), 1)
        return _greedy(act_ref, extf_ref, bx_ref[0:1, :], bx_ref[1:2, :],
                       bx_ref[2:3, :], bx_ref[3:4, :], lane, C, NPAD)

    kept_idx, kept_val, kept_sc = lax.cond(fb, full_fn, lambda _: kept_hot,
                                           None)
    _merge(kept_idx, kept_val, kept_sc, idx_out, lab_out, valid_out,
           score_out)


def _run_hot(hsc, hidx, hx1, hy1, hx2, hy2, cnt128, cls_t, bx):
    C, NPAD = cls_t.shape
    return pl.pallas_call(
        _hot_body,
        out_shape=[
            jax.ShapeDtypeStruct((1, 128), jnp.int32),
            jax.ShapeDtypeStruct((1, 128), jnp.int32),
            jax.ShapeDtypeStruct((1, 128), jnp.int32),
            jax.ShapeDtypeStruct((1, 128), jnp.float32),
        ],
        scratch_shapes=[pltpu.VMEM((C, HOTW), jnp.float32),
                        pltpu.VMEM((C, NPAD), jnp.float32),
                        pltpu.VMEM((5 * C, HOTW), jnp.float32),
                        pltpu.VMEM((5 * C, NPAD), jnp.float32)],
    )(hsc, hidx, hx1, hy1, hx2, hy2, cnt128, cls_t, bx)


def _make_sc_gather(n, d_feat):
    info = plsc.get_sparse_core_info()
    nc = info.num_cores
    mesh = plsc.VectorSubcoreMesh(core_axis_name="c", subcore_axis_name="s")

    @functools.partial(
        pl.kernel, mesh=mesh,
        compiler_params=pltpu.CompilerParams(use_tc_tiling_on_sc=False),
        out_type=jax.ShapeDtypeStruct((128, d_feat), jnp.float32),
        scratch_types=[
            pltpu.VMEM((32,), jnp.int32),
            pltpu.VMEM((32, d_feat), jnp.float32),
            pltpu.SemaphoreType.DMA,
        ],
    )
    def gather_k(idx_hbm, feat_hbm, out_feat, idx_v, feat_v, sem):
        wid = lax.axis_index("s") * nc + lax.axis_index("c")

        @pl.when(wid < 4)
        def _():
            pltpu.sync_copy(idx_hbm.at[pl.ds(wid * 32, 32)], idx_v)
            pltpu.async_copy(feat_hbm.at[idx_v], feat_v, sem).wait()
            pltpu.sync_copy(feat_v, out_feat.at[pl.ds(wid * 32, 32)])

    return gather_k


def kernel(boxes, classification, rotation, translation, hand):
    b = boxes[0]
    c = classification[0]
    r = rotation[0]
    t = translation[0]
    h = hand[0]
    n, nclass = c.shape
    npad = ((n + 127) // 128) * 128

    cls_t = jnp.pad(c.T, ((0, 0), (0, npad - n)), constant_values=-1.0)
    bx = jnp.pad(b.T, ((0, 8 - b.shape[1]), (0, npad - n)))

    tau128, cnt128 = _run_tau(cls_t)
    tau16 = jnp.pad(tau128[:, 0], (0, 16 - nclass))
    hsc, hidx, hx1, hy1, hx2, hy2 = _make_sc_compact(nclass, npad)(
        cls_t, bx, tau16)
    sidx, slab, sval, ssc = _run_hot(hsc, hidx, hx1, hy1, hx2, hy2,
                                     cnt128, cls_t, bx)
    idx128 = sidx[0]

    db, dr, dt, dh = b.shape[1], r.shape[1], t.shape[1], h.shape[1]
    d_used = db + dr + dt + dh
    d_feat = ((d_used + 15) // 16) * 16
    feat = jnp.concatenate(
        [b, r, t, h, jnp.zeros((n, d_feat - d_used), jnp.float32)], axis=1)
    g = _make_sc_gather(n, d_feat)(idx128, feat)

    m = MAX_DETECTIONS
    valid = sval[0, :m] > 0
    out_boxes = jnp.where(valid[:, None], g[:m, :db], -1.0)
    out_scores = jnp.where(valid, ssc[0, :m], -1.0)
    out_labels = jnp.where(valid, slab[0, :m], -1).astype(jnp.int32)
    out_rot = jnp.where(valid[:, None], g[:m, db:db + dr], -1.0)
    out_tr = jnp.where(valid[:, None], g[:m, db + dr:db + dr + dt], -1.0)
    out_hand = jnp.where(valid[:, None], g[:m, db + dr + dt:d_used], -1.0)
    return (out_boxes, out_scores, out_labels, out_rot, out_tr, out_hand)


# speedup vs baseline: 3.5761x; 1.1833x over previous
import functools

import jax
import jax.numpy as jnp
from jax import lax
from jax.experimental import pallas as pl
from jax.experimental.pallas import tpu as pltpu
from jax.experimental.pallas import tpu_sc as plsc

SCORE_THRESHOLD = 0.01
NMS_THRESHOLD = 0.5
MAX_DETECTIONS = 100
NEG = -1e30
CAP = 192
HOTW = 2 * CAP


def _greedy(active_ref, ext_ref, x1, y1, x2, y2, idx_of_lane, n_classes,
            width):
    C, W = n_classes, width
    areas = (x2 - x1) * (y2 - y1)
    lane = lax.broadcasted_iota(jnp.int32, (C, W), 1)
    lane_k = lax.broadcasted_iota(jnp.int32, (C, 128), 1)
    BIG = jnp.int32(2**30)
    kept0 = (jnp.zeros((C, 128), jnp.int32), jnp.zeros((C, 128), jnp.int32),
             jnp.full((C, 128), NEG, jnp.float32))

    def nms_iter(it, carry):
        kept_idx, kept_val, kept_sc = carry
        act = active_ref[...]
        m = jnp.max(act, axis=1, keepdims=True)
        ismax = act == m
        psel = jnp.min(jnp.where(ismax, lane, BIG), axis=1, keepdims=True)
        onehot = lane == psel
        valid = m > -1e20
        ext_ref[0 * C:1 * C, :] = jnp.where(onehot, x1, NEG)
        ext_ref[1 * C:2 * C, :] = jnp.where(onehot, y1, NEG)
        ext_ref[2 * C:3 * C, :] = jnp.where(onehot, x2, NEG)
        ext_ref[3 * C:4 * C, :] = jnp.where(onehot, y2, NEG)
        ext_ref[4 * C:5 * C, :] = jnp.where(
            onehot, idx_of_lane.astype(jnp.float32), NEG)
        red = jnp.max(ext_ref[...], axis=1, keepdims=True)
        x1s = red[0 * C:1 * C]
        y1s = red[1 * C:2 * C]
        x2s = red[2 * C:3 * C]
        y2s = red[3 * C:4 * C]
        isel = jnp.where(valid, red[4 * C:5 * C], 0.0).astype(jnp.int32)
        area_s = (x2s - x1s) * (y2s - y1s)
        w = jnp.maximum(0.0, jnp.minimum(x2s, x2) - jnp.maximum(x1s, x1))
        h = jnp.maximum(0.0, jnp.minimum(y2s, y2) - jnp.maximum(y1s, y1))
        inter = w * h
        iou = inter / (area_s + areas - inter + 1e-9)
        suppress = valid & ((iou > NMS_THRESHOLD) | onehot)
        active_ref[...] = jnp.where(suppress, NEG, act)
        slot = lane_k == it
        kept_idx = jnp.where(slot, jnp.where(valid, isel, 0), kept_idx)
        kept_val = jnp.where(slot, valid.astype(jnp.int32), kept_val)
        kept_sc = jnp.where(slot, m, kept_sc)
        return kept_idx, kept_val, kept_sc

    return lax.fori_loop(0, MAX_DETECTIONS, nms_iter, kept0)


def _merge(kept_idx, kept_val, kept_sc, idx_out, lab_out, valid_out,
           score_out):
    C = kept_idx.shape[0]
    row_i = lax.broadcasted_iota(jnp.int32, (128, 128), 0)
    col_i = lax.broadcasted_iota(jnp.int32, (128, 128), 1)
    eye = row_i == col_i
    lane_t = lax.broadcasted_iota(jnp.int32, (1, 128), 1)
    rcol = lax.broadcasted_iota(jnp.int32, (128, 1), 0)

    sbits = lax.bitcast_convert_type(kept_sc, jnp.int32)
    acc = jnp.zeros((10, 128), jnp.float32)
    for cc in range(C):
        srow_c = kept_sc[cc:cc + 1, :]
        scol = jnp.max(jnp.where(eye, jnp.broadcast_to(srow_c, (128, 128)),
                                 NEG), axis=1, keepdims=True)
        fcol = cc * 128 + rcol
        pre = jnp.zeros((128, 128), jnp.int32)
        for c2 in range(C):
            srow = kept_sc[c2:c2 + 1, :]
            frow = c2 * 128 + lane_t
            pre = pre + ((srow > scol)
                         | ((srow == scol) & (frow < fcol))).astype(jnp.int32)
        rank_c = jnp.sum(pre, axis=1, keepdims=True)
        onehot = (rank_c == lane_t).astype(jnp.float32)
        u = sbits[cc:cc + 1, :]
        iv = kept_idx[cc:cc + 1, :]
        rows = [lax.shift_right_logical(u, 7 * k) & 127 for k in range(4)]
        rows.append(lax.shift_right_logical(u, 28) & 15)
        rows.append(iv & 127)
        rows.append(lax.shift_right_logical(iv, 7) & 127)
        rows.append(lax.shift_right_logical(iv, 14) & 127)
        rows.append(kept_val[cc:cc + 1, :])
        rows.append(jnp.full((1, 128), cc, jnp.int32))
        vals = jnp.concatenate([r.astype(jnp.float32) for r in rows], axis=0)
        acc = acc + jnp.dot(vals, onehot,
                            preferred_element_type=jnp.float32)
    d = (acc + 0.5).astype(jnp.int32)
    sb = (d[0:1] | (d[1:2] << 7) | (d[2:3] << 14) | (d[3:4] << 21)
          | (d[4:5] << 28))
    score_out[...] = lax.bitcast_convert_type(sb, jnp.float32)
    idx_out[...] = d[5:6] | (d[6:7] << 7) | (d[7:8] << 14)
    valid_out[...] = d[8:9]
    lab_out[...] = d[9:10]


def _tau_body(cls_ref, tau_out, cnt_out, act_ref):
    C, NPAD = cls_ref.shape
    cls = cls_ref[...]
    act = jnp.where(cls > SCORE_THRESHOLD, cls, NEG)
    act_ref[...] = act
    cntall = jnp.sum((act > -1e20).astype(jnp.int32), axis=1, keepdims=True)
    mx = jnp.max(act, axis=1, keepdims=True)
    lo0 = jnp.full((C, 1), SCORE_THRESHOLD, jnp.float32)
    hi0 = jnp.maximum(mx, lo0)

    def step(_, carry):
        lo, hi = carry
        mid = 0.5 * (lo + hi)
        a = act_ref[...]
        cnt = jnp.sum((a > mid).astype(jnp.int32), axis=1, keepdims=True)
        over = cnt > CAP
        return jnp.where(over, mid, lo), jnp.where(over, hi, mid)

    _, hi = lax.fori_loop(0, 20, step, (lo0, hi0))
    tau_out[...] = jnp.broadcast_to(hi, (C, 128))
    cnt_out[...] = jnp.broadcast_to(cntall, (C, 128))


def _run_tau(cls_t):
    C, NPAD = cls_t.shape
    return pl.pallas_call(
        _tau_body,
        out_shape=[
            jax.ShapeDtypeStruct((C, 128), jnp.float32),
            jax.ShapeDtypeStruct((C, 128), jnp.int32),
        ],
        scratch_shapes=[pltpu.VMEM((C, NPAD), jnp.float32)],
    )(cls_t)


def _make_sc_compact(n_classes, npad):
    half = npad // 2
    nv = half // 16
    info = plsc.get_sparse_core_info()
    nc = info.num_cores
    mesh = plsc.VectorSubcoreMesh(core_axis_name="c", subcore_axis_name="s")
    C = n_classes

    @functools.partial(
        pl.kernel, mesh=mesh,
        compiler_params=pltpu.CompilerParams(use_tc_tiling_on_sc=False,
                                             needs_layout_passes=False),
        out_type=[
            jax.ShapeDtypeStruct((C, HOTW), jnp.float32),
            jax.ShapeDtypeStruct((C, HOTW), jnp.int32),
            jax.ShapeDtypeStruct((C, HOTW), jnp.float32),
            jax.ShapeDtypeStruct((C, HOTW), jnp.float32),
            jax.ShapeDtypeStruct((C, HOTW), jnp.float32),
            jax.ShapeDtypeStruct((C, HOTW), jnp.float32),
        ],
        scratch_types=[
            pltpu.VMEM((half,), jnp.float32),
            pltpu.VMEM((half,), jnp.float32),
            pltpu.VMEM((half,), jnp.float32),
            pltpu.VMEM((half,), jnp.float32),
            pltpu.VMEM((half,), jnp.float32),
            pltpu.VMEM((16,), jnp.float32),
            pltpu.VMEM((CAP,), jnp.float32),
            pltpu.VMEM((CAP,), jnp.int32),
            pltpu.VMEM((CAP,), jnp.float32),
            pltpu.VMEM((CAP,), jnp.float32),
            pltpu.VMEM((CAP,), jnp.float32),
            pltpu.VMEM((CAP,), jnp.float32),
        ],
    )
    def compact_k(cls_hbm, bx_hbm, tau_hbm,
                  hsc_hbm, hidx_hbm, hx1_hbm, hy1_hbm, hx2_hbm, hy2_hbm,
                  srow, rx1, ry1, rx2, ry2, tau_v,
                  osc, oidx, ox1, oy1, ox2, oy2):
        wid = lax.axis_index("s") * nc + lax.axis_index("c")

        @pl.when(wid < 2 * C)
        def _():
            c = wid // 2
            hf = wid % 2
            base = hf * half
            pltpu.sync_copy(cls_hbm.at[c, pl.ds(base, half)], srow)
            pltpu.sync_copy(bx_hbm.at[0, pl.ds(base, half)], rx1)
            pltpu.sync_copy(bx_hbm.at[1, pl.ds(base, half)], ry1)
            pltpu.sync_copy(bx_hbm.at[2, pl.ds(base, half)], rx2)
            pltpu.sync_copy(bx_hbm.at[3, pl.ds(base, half)], ry2)
            pltpu.sync_copy(tau_hbm, tau_v)

            iota16 = lax.iota(jnp.int32, 16)
            zf = jnp.zeros((16,), jnp.float32)
            for k in range(CAP // 16):
                osc[pl.ds(k * 16, 16)] = zf + NEG
                oidx[pl.ds(k * 16, 16)] = iota16 * 0
                ox1[pl.ds(k * 16, 16)] = zf
                oy1[pl.ds(k * 16, 16)] = zf
                ox2[pl.ds(k * 16, 16)] = zf
                oy2[pl.ds(k * 16, 16)] = zf

            tau_c = plsc.load_gather(tau_v, [iota16 * 0 + c])
            base16 = iota16 + base

            def body(j, cnt):
                for u in range(4):
                    o = j * 64 + u * 16
                    s = srow[pl.ds(o, 16)]
                    mask = s > tau_c
                    pos = cnt + plsc.cumsum(mask.astype(jnp.int32)) - 1
                    wr = mask & (pos < CAP)
                    plsc.store_scatter(osc, [pos], s, mask=wr)
                    plsc.store_scatter(oidx, [pos], base16 + o, mask=wr)
                    plsc.store_scatter(ox1, [pos], rx1[pl.ds(o, 16)],
                                       mask=wr)
                    plsc.store_scatter(oy1, [pos], ry1[pl.ds(o, 16)],
                                       mask=wr)
                    plsc.store_scatter(ox2, [pos], rx2[pl.ds(o, 16)],
                                       mask=wr)
                    plsc.store_scatter(oy2, [pos], ry2[pl.ds(o, 16)],
                                       mask=wr)
                    cnt = cnt + plsc.all_reduce_population_count(mask)
                return cnt

            cnt = lax.fori_loop(0, nv // 4, body,
                                jnp.zeros((16,), jnp.int32))
            for jt in range(4 * (nv // 4), nv):
                o = jt * 16
                s = srow[pl.ds(o, 16)]
                mask = s > tau_c
                pos = cnt + plsc.cumsum(mask.astype(jnp.int32)) - 1
                wr = mask & (pos < CAP)
                plsc.store_scatter(osc, [pos], s, mask=wr)
                plsc.store_scatter(oidx, [pos], base16 + o, mask=wr)
                plsc.store_scatter(ox1, [pos], rx1[pl.ds(o, 16)], mask=wr)
                plsc.store_scatter(oy1, [pos], ry1[pl.ds(o, 16)], mask=wr)
                plsc.store_scatter(ox2, [pos], rx2[pl.ds(o, 16)], mask=wr)
                plsc.store_scatter(oy2, [pos], ry2[pl.ds(o, 16)], mask=wr)
                cnt = cnt + plsc.all_reduce_population_count(mask)

            hout = hf * CAP
            pltpu.sync_copy(osc, hsc_hbm.at[c, pl.ds(hout, CAP)])
            pltpu.sync_copy(oidx, hidx_hbm.at[c, pl.ds(hout, CAP)])
            pltpu.sync_copy(ox1, hx1_hbm.at[c, pl.ds(hout, CAP)])
            pltpu.sync_copy(oy1, hy1_hbm.at[c, pl.ds(hout, CAP)])
            pltpu.sync_copy(ox2, hx2_hbm.at[c, pl.ds(hout, CAP)])
            pltpu.sync_copy(oy2, hy2_hbm.at[c, pl.ds(hout, CAP)])

    return compact_k


def _hot_body(hsc_ref, hidx_ref, hx1_ref, hy1_ref, hx2_ref, hy2_ref,
              cnt_ref, cls_ref, bx_ref,
              idx_out, lab_out, valid_out, score_out, hact_ref, act_ref,
              exth_ref, extf_ref):
    C, NPAD = cls_ref.shape
    hact_ref[...] = hsc_ref[...]
    w_hot = jnp.sum((hsc_ref[...] > -1e20).astype(jnp.int32), axis=1,
                    keepdims=True)
    kept_hot = _greedy(hact_ref, exth_ref, hx1_ref[...], hy1_ref[...],
                       hx2_ref[...], hy2_ref[...], hidx_ref[...], C, HOTW)
    nkept = jnp.sum(kept_hot[1], axis=1, keepdims=True)
    need_fb = (nkept < MAX_DETECTIONS) & (cnt_ref[:, 0:1] > w_hot)
    fb = jnp.max(need_fb.astype(jnp.int32)) > 0

    def full_fn(_):
        cls = cls_ref[...]
        act_ref[...] = jnp.where(cls > SCORE_THRESHOLD, cls, NEG)
        lane = lax.broadcasted_iota(jnp.int32, (C, NPAD), 1)
        return _greedy(act_ref, extf_ref, bx_ref[0:1, :], bx_ref[1:2, :],
                       bx_ref[2:3, :], bx_ref[3:4, :], lane, C, NPAD)

    kept_idx, kept_val, kept_sc = lax.cond(fb, full_fn, lambda _: kept_hot,
                                           None)
    _merge(kept_idx, kept_val, kept_sc, idx_out, lab_out, valid_out,
           score_out)


def _run_hot(hsc, hidx, hx1, hy1, hx2, hy2, cnt128, cls_t, bx):
    C, NPAD = cls_t.shape
    return pl.pallas_call(
        _hot_body,
        out_shape=[
            jax.ShapeDtypeStruct((1, 128), jnp.int32),
            jax.ShapeDtypeStruct((1, 128), jnp.int32),
            jax.ShapeDtypeStruct((1, 128), jnp.int32),
            jax.ShapeDtypeStruct((1, 128), jnp.float32),
        ],
        scratch_shapes=[pltpu.VMEM((C, HOTW), jnp.float32),
                        pltpu.VMEM((C, NPAD), jnp.float32),
                        pltpu.VMEM((5 * C, HOTW), jnp.float32),
                        pltpu.VMEM((5 * C, NPAD), jnp.float32)],
    )(hsc, hidx, hx1, hy1, hx2, hy2, cnt128, cls_t, bx)


def _make_sc_gather(n, d_feat):
    info = plsc.get_sparse_core_info()
    nc = info.num_cores
    mesh = plsc.VectorSubcoreMesh(core_axis_name="c", subcore_axis_name="s")

    @functools.partial(
        pl.kernel, mesh=mesh,
        compiler_params=pltpu.CompilerParams(use_tc_tiling_on_sc=False),
        out_type=jax.ShapeDtypeStruct((128, d_feat), jnp.float32),
        scratch_types=[
            pltpu.VMEM((32,), jnp.int32),
            pltpu.VMEM((32, d_feat), jnp.float32),
            pltpu.SemaphoreType.DMA,
        ],
    )
    def gather_k(idx_hbm, feat_hbm, out_feat, idx_v, feat_v, sem):
        wid = lax.axis_index("s") * nc + lax.axis_index("c")

        @pl.when(wid < 4)
        def _():
            pltpu.sync_copy(idx_hbm.at[pl.ds(wid * 32, 32)], idx_v)
            pltpu.async_copy(feat_hbm.at[idx_v], feat_v, sem).wait()
            pltpu.sync_copy(feat_v, out_feat.at[pl.ds(wid * 32, 32)])

    return gather_k


def kernel(boxes, classification, rotation, translation, hand):
    b = boxes[0]
    c = classification[0]
    r = rotation[0]
    t = translation[0]
    h = hand[0]
    n, nclass = c.shape
    npad = ((n + 127) // 128) * 128

    cls_t = jnp.pad(c.T, ((0, 0), (0, npad - n)), constant_values=-1.0)
    bx = jnp.pad(b.T, ((0, 8 - b.shape[1]), (0, npad - n)))

    tau128, cnt128 = _run_tau(cls_t)
    tau16 = jnp.pad(tau128[:, 0], (0, 16 - nclass))
    hsc, hidx, hx1, hy1, hx2, hy2 = _make_sc_compact(nclass, npad)(
        cls_t, bx, tau16)
    sidx, slab, sval, ssc = _run_hot(hsc, hidx, hx1, hy1, hx2, hy2,
                                     cnt128, cls_t, bx)
    idx128 = sidx[0]

    db, dr, dt, dh = b.shape[1], r.shape[1], t.shape[1], h.shape[1]
    d_used = db + dr + dt + dh
    d_feat = ((d_used + 15) // 16) * 16
    feat = jnp.concatenate(
        [b, r, t, h, jnp.zeros((n, d_feat - d_used), jnp.float32)], axis=1)
    g = _make_sc_gather(n, d_feat)(idx128, feat)

    m = MAX_DETECTIONS
    valid = sval[0, :m] > 0
    out_boxes = jnp.where(valid[:, None], g[:m, :db], -1.0)
    out_scores = jnp.where(valid, ssc[0, :m], -1.0)
    out_labels = jnp.where(valid, slab[0, :m], -1).astype(jnp.int32)
    out_rot = jnp.where(valid[:, None], g[:m, db:db + dr], -1.0)
    out_tr = jnp.where(valid[:, None], g[:m, db + dr:db + dr + dt], -1.0)
    out_hand = jnp.where(valid[:, None], g[:m, db + dr + dt:d_used], -1.0)
    return (out_boxes, out_scores, out_labels, out_rot, out_tr, out_hand)
